# baseline (speedup 1.0000x reference)
import functools

import jax
import jax.numpy as jnp
from jax import lax
from jax.experimental import pallas as pl
from jax.experimental.pallas import tpu as pltpu
from jax.experimental.pallas import tpu_sc as plsc

N1 = 50000
N2 = 50000
E = 800000
NSTATE = 64
HEADS = 3
NTYPES = 3
NSTEP = 2
HID = HEADS * NSTATE

_BLK = 2000

_NSC = 2
_NTILE = 16
_R = 4576
_NCHUNK_PER_SC = 6
_OUT_N = _NSC * _NCHUNK_PER_SC * _R
_ROW = 208
_TS = E // _NTILE
_SR = 2000
_LD = 2000
_GB = 128
_CAP = 2176


def _mlp_block(x, W0, b0, W1, b1, W2, b2):
    h = jnp.maximum(x @ W0 + b0, 0.0)
    h = jnp.maximum(h @ W1 + b1, 0.0)
    return h @ W2 + b2


def _full(a):
    return pl.BlockSpec(a.shape, lambda i: (0,) * a.ndim)


def _mlp_kernel_body(x_ref, W0, b0, W1, b1, W2, b2, o_ref):
    o_ref[...] = _mlp_block(x_ref[...], W0[...], b0[...], W1[...], b1[...],
                            W2[...], b2[...])


def _mlp_pallas(p, x, nout):
    n = x.shape[0]
    args = (p["W0"], p["b0"], p["W1"], p["b1"], p["W2"], p["b2"])
    return pl.pallas_call(
        _mlp_kernel_body,
        grid=(n // _BLK,),
        in_specs=[pl.BlockSpec((_BLK, x.shape[1]), lambda i: (i, 0))]
        + [_full(a) for a in args],
        out_specs=pl.BlockSpec((_BLK, nout), lambda i: (i, 0)),
        out_shape=jax.ShapeDtypeStruct((n, nout), jnp.float32),
    )(x, *args)


def _enc1_body(x_ref, *rest):
    o_ref = rest[-1]
    x = x_ref[...]
    types = x[:, 0].astype(jnp.int32)
    out = jnp.zeros((x.shape[0], NSTATE), jnp.float32)
    for i in range(NTYPES):
        Wi = rest[i * 6:(i + 1) * 6]
        yi = _mlp_block(x, *[w[...] for w in Wi])
        out = jnp.where((types == i)[:, None], yi, out)
    o_ref[...] = out


def _enc1_pallas(ps, x):
    args = []
    for p in ps:
        args += [p["W0"], p["b0"], p["W1"], p["b1"], p["W2"], p["b2"]]
    return pl.pallas_call(
        _enc1_body,
        grid=(x.shape[0] // _BLK,),
        in_specs=[pl.BlockSpec((_BLK, x.shape[1]), lambda i: (i, 0))]
        + [_full(a) for a in args],
        out_specs=pl.BlockSpec((_BLK, NSTATE), lambda i: (i, 0)),
        out_shape=jax.ShapeDtypeStruct((x.shape[0], NSTATE), jnp.float32),
    )(x, *args)


def _proj_body(x_ref, Ws, Asrc, hs_ref, a_ref):
    hs = x_ref[...] @ Ws[...]
    hs_ref[...] = hs
    a_ref[...] = hs @ Asrc[...]


def _proj_pallas(x, Ws, Asrc):
    n = x.shape[0]
    return pl.pallas_call(
        _proj_body,
        grid=(n // _BLK,),
        in_specs=[pl.BlockSpec((_BLK, x.shape[1]), lambda i: (i, 0)),
                  _full(Ws), _full(Asrc)],
        out_specs=(pl.BlockSpec((_BLK, HID), lambda i: (i, 0)),
                   pl.BlockSpec((_BLK, 16), lambda i: (i, 0))),
        out_shape=(jax.ShapeDtypeStruct((n, HID), jnp.float32),
                   jax.ShapeDtypeStruct((n, 16), jnp.float32)),
    )(x, Ws, Asrc)


def _adst_body(x_ref, W, a_ref):
    a_ref[...] = x_ref[...] @ W[...]


def _adst_pallas(x, W):
    n = x.shape[0]
    return pl.pallas_call(
        _adst_body,
        grid=(n // _BLK,),
        in_specs=[pl.BlockSpec((_BLK, x.shape[1]), lambda i: (i, 0)), _full(W)],
        out_specs=pl.BlockSpec((_BLK, 16), lambda i: (i, 0)),
        out_shape=jax.ShapeDtypeStruct((n, 16), jnp.float32),
    )(x, W)


def _adst_sel_body(x_ref, t_ref, *rest):
    o_ref = rest[-1]
    x = x_ref[...]
    types = t_ref[...][:, 0]
    out = jnp.zeros((x.shape[0], 16), jnp.float32)
    for i in range(NTYPES):
        W, tv = rest[2 * i], rest[2 * i + 1]
        yi = x @ W[...] + tv[...]
        out = jnp.where((types == i)[:, None], yi, out)
    o_ref[...] = out


def _adst_sel_pallas(x, types2d, Ws, tvs):
    n = x.shape[0]
    args = []
    for W, tv in zip(Ws, tvs):
        args += [W, tv]
    return pl.pallas_call(
        _adst_sel_body,
        grid=(n // _BLK,),
        in_specs=[pl.BlockSpec((_BLK, x.shape[1]), lambda i: (i, 0)),
                  pl.BlockSpec((_BLK, 1), lambda i: (i, 0))]
        + [_full(a) for a in args],
        out_specs=pl.BlockSpec((_BLK, 16), lambda i: (i, 0)),
        out_shape=jax.ShapeDtypeStruct((n, 16), jnp.float32),
    )(x, types2d, *args)


def _gru_block(x, h, WihT, bih, WhhT, bhh):
    gi = x @ WihT + bih
    gh = h @ WhhT + bhh
    ir, iz, i_n = gi[:, :NSTATE], gi[:, NSTATE:2 * NSTATE], gi[:, 2 * NSTATE:]
    hr, hz, h_n = gh[:, :NSTATE], gh[:, NSTATE:2 * NSTATE], gh[:, 2 * NSTATE:]
    r = jax.nn.sigmoid(ir + hr)
    z = jax.nn.sigmoid(iz + hz)
    n = jnp.tanh(i_n + r * h_n)
    return (1.0 - z) * n + z * h


def _gru_body(x_ref, h_ref, WihT, bih, WhhT, bhh, o_ref):
    o_ref[...] = _gru_block(x_ref[...], h_ref[...], WihT[...], bih[...],
                            WhhT[...], bhh[...])


def _gru_pallas(p, x, h):
    n = x.shape[0]
    args = (p["WihT"], p["bih"], p["WhhT"], p["bhh"])
    return pl.pallas_call(
        _gru_body,
        grid=(n // _BLK,),
        in_specs=[pl.BlockSpec((_BLK, NSTATE), lambda i: (i, 0))] * 2
        + [_full(a) for a in args],
        out_specs=pl.BlockSpec((_BLK, NSTATE), lambda i: (i, 0)),
        out_shape=jax.ShapeDtypeStruct((n, NSTATE), jnp.float32),
    )(x, h, *args)


def _gru1_body(x_ref, h_ref, t_ref, *rest):
    o_ref = rest[-1]
    x, h = x_ref[...], h_ref[...]
    types = t_ref[...][:, 0]
    out = jnp.zeros_like(h)
    for i in range(NTYPES):
        Wi = [w[...] for w in rest[i * 4:(i + 1) * 4]]
        yi = _gru_block(x, h, *Wi)
        out = jnp.where((types == i)[:, None], yi, out)
    o_ref[...] = out


def _gru1_pallas(ps, x, h, types2d):
    n = x.shape[0]
    args = []
    for p in ps:
        args += [p["WihT"], p["bih"], p["WhhT"], p["bhh"]]
    return pl.pallas_call(
        _gru1_body,
        grid=(n // _BLK,),
        in_specs=[pl.BlockSpec((_BLK, NSTATE), lambda i: (i, 0))] * 2
        + [pl.BlockSpec((_BLK, 1), lambda i: (i, 0))]
        + [_full(a) for a in args],
        out_specs=pl.BlockSpec((_BLK, NSTATE), lambda i: (i, 0)),
        out_shape=jax.ShapeDtypeStruct((n, NSTATE), jnp.float32),
    )(x, h, types2d, *args)


def _pool_norm_body(acc_ref, bias_ref, Q_ref, P_ref, o_ref):
    acc = acc_ref[...]
    agg = acc[:, :HID]
    den = acc[:, HID:HID + HEADS]
    den192 = den @ Q_ref[...]
    m = agg / (den192 + 1e-30) + bias_ref[...]
    o_ref[...] = m @ P_ref[...]


def _pool_norm_pallas(acc, bias, Q, P):
    return pl.pallas_call(
        _pool_norm_body,
        grid=(N2 // _BLK,),
        in_specs=[pl.BlockSpec((_BLK, _ROW), lambda i: (i, 0)),
                  _full(bias), _full(Q), _full(P)],
        out_specs=pl.BlockSpec((_BLK, NSTATE), lambda i: (i, 0)),
        out_shape=jax.ShapeDtypeStruct((N2, NSTATE), jnp.float32),
    )(acc, bias, Q, P)


def _pool_norm_sel_body(acc_ref, t_ref, b0, b1, b2, Q_ref, P_ref, o_ref):
    acc = acc_ref[...]
    types = t_ref[...][:, 0]
    agg = acc[:, :HID]
    den = acc[:, HID:HID + HEADS]
    den192 = den @ Q_ref[...]
    bias = jnp.where((types == 0)[:, None], b0[...],
                     jnp.where((types == 1)[:, None], b1[...], b2[...]))
    m = agg / (den192 + 1e-30) + bias
    o_ref[...] = m @ P_ref[...]


def _pool_norm_sel_pallas(acc, types2d, biases, Q, P):
    return pl.pallas_call(
        _pool_norm_sel_body,
        grid=(N1 // _BLK,),
        in_specs=[pl.BlockSpec((_BLK, _ROW), lambda i: (i, 0)),
                  pl.BlockSpec((_BLK, 1), lambda i: (i, 0))]
        + [_full(b) for b in biases] + [_full(Q), _full(P)],
        out_specs=pl.BlockSpec((_BLK, NSTATE), lambda i: (i, 0)),
        out_shape=jax.ShapeDtypeStruct((N1, NSTATE), jnp.float32),
    )(acc, types2d, *biases, Q, P)


def _edge_sc(srcs, dsts, hs, asrc, adst_pad, csplat, tstride):
    mesh = plsc.VectorSubcoreMesh(core_axis_name="c", subcore_axis_name="s",
                                  num_cores=_NSC, num_subcores=_NTILE)

    @functools.partial(
        pl.kernel,
        out_type=jax.ShapeDtypeStruct((_OUT_N, _ROW), jnp.float32),
        mesh=mesh,
        compiler_params=pltpu.CompilerParams(
            needs_layout_passes=False, use_tc_tiling_on_sc=False),
        scratch_types=[
            pltpu.VMEM((_CAP,), jnp.int32),
            pltpu.VMEM((_CAP,), jnp.int32),
            pltpu.VMEM((_LD,), jnp.int32),
            pltpu.VMEM((_LD,), jnp.int32),
            pltpu.VMEM((_GB, HID), jnp.float32),
            pltpu.VMEM((_GB, 16), jnp.float32),
            pltpu.VMEM((_GB, 16), jnp.float32),
            pltpu.VMEM((_GB, _ROW), jnp.float32),
            pltpu.VMEM((HEADS, _GB), jnp.float32),
            pltpu.VMEM((_GB,), jnp.int32),
            pltpu.VMEM((_GB,), jnp.int32),
            pltpu.VMEM((_GB,), jnp.int32),
            pltpu.VMEM((8, _ROW), jnp.float32),
            pltpu.VMEM((HEADS, 16), jnp.float32),
            pltpu.VMEM_SHARED((_R + 16, _ROW), jnp.float32),
            pltpu.SemaphoreType.DMA,
            pltpu.SemaphoreType.DMA,
            pltpu.SemaphoreType.DMA,
        ],
    )
    def k(srcs_h, dsts_h, hs_h, asrc_h, adst_h, c_h, out_h,
          sC, dC, sL, dL, hsr, asr, adr, stg, exb, gidx, fidx, lidx,
          zbuf, cbuf, acc, sem1, sem2, sem3):
        cid = lax.axis_index("c")
        sid = lax.axis_index("s")
        ebase = sid * _TS

        pltpu.sync_copy(c_h, cbuf)
        for r in range(8):
            for g in range(_ROW // 16):
                zbuf[r, pl.ds(g * 16, 16)] = jnp.zeros((16,), jnp.float32)

        for j in range(_NCHUNK_PER_SC):
            cbase = (cid * _NCHUNK_PER_SC + j) * _R

            nzb = (_R + 16) // 8

            def zero_body(kk, _):
                b = sid + kk * _NTILE

                @pl.when(b < nzb)
                def _():
                    pltpu.sync_copy(zbuf, acc.at[pl.ds(b * 8, 8)])
                return 0

            lax.fori_loop(0, (nzb + _NTILE - 1) // _NTILE, zero_body, 0)
            plsc.subcore_barrier()

            def sub_body(sub, _):
                sbase = ebase + sub * _SR

                def load_body(b, cur):
                    pltpu.sync_copy(srcs_h.at[pl.ds(sbase + b * _LD, _LD)], sL)
                    pltpu.sync_copy(dsts_h.at[pl.ds(sbase + b * _LD, _LD)], dL)

                    def scan_body(g, cur):
                        s16 = sL[pl.ds(g * 16, 16)]
                        d16 = dL[pl.ds(g * 16, 16)]
                        loc = d16 - jnp.full((16,), cbase, jnp.int32)
                        own = (loc >= 0) & (loc < _R)
                        inc = plsc.cumsum(own.astype(jnp.int32))
                        pos = jnp.maximum(cur + inc - 1, 0)
                        plsc.store_scatter(sC, [pos], s16, mask=own)
                        plsc.store_scatter(dC, [pos], loc, mask=own)
                        return cur + jnp.sum(own.astype(jnp.int32))

                    return lax.fori_loop(0, _LD // 16, scan_body, cur)

                cur = lax.fori_loop(0, _SR // _LD, load_body, jnp.int32(0))

                nb = (cur + _GB - 1) // _GB
                for kk in range(_GB // 16):
                    idx = cur + kk * 16 + lax.iota(jnp.int32, 16)
                    pm = idx < nb * _GB
                    plsc.store_scatter(sC, [idx], jnp.zeros((16,), jnp.int32),
                                       mask=pm)
                    plsc.store_scatter(dC, [idx],
                                       jnp.full((16,), _R, jnp.int32), mask=pm)

                def batch_body(bi, _):
                    off = bi * _GB

                    def idx_body(q, _):
                        sv = sC[pl.ds(off + q * 16, 16)]
                        lv = dC[pl.ds(off + q * 16, 16)]
                        fidx[pl.ds(q * 16, 16)] = jnp.minimum(
                            lv + jnp.full((16,), cbase, jnp.int32),
                            jnp.full((16,), N1 - 1, jnp.int32))
                        lidx[pl.ds(q * 16, 16)] = lv
                        gidx[pl.ds(q * 16, 16)] = sv
                        return 0

                    lax.fori_loop(0, _GB // 16, idx_body, 0)
                    pltpu.async_copy(adst_h.at[fidx], adr, sem1).wait()

                    if tstride:
                        def tmix_body(q, _):
                            t = plsc.load_gather(
                                adr,
                                [q * 16 + lax.iota(jnp.int32, 16),
                                 jnp.full((16,), 3, jnp.int32)])
                            ti = t.astype(jnp.int32)
                            sv = gidx[pl.ds(q * 16, 16)]
                            gidx[pl.ds(q * 16, 16)] = ti * tstride + sv
                            return 0

                        lax.fori_loop(0, _GB // 16, tmix_body, 0)

                    c1 = pltpu.async_copy(asrc_h.at[gidx], asr, sem2)
                    c2 = pltpu.async_copy(hs_h.at[gidx], hsr, sem3)
                    c1.wait()
                    c2.wait()

                    for h in range(HEADS):
                        def att_body(q, _):
                            ch = cbuf[h, :]
                            rows = q * 16 + lax.iota(jnp.int32, 16)
                            col = jnp.full((16,), h, jnp.int32)
                            asv = plsc.load_gather(asr, [rows, col])
                            adv = plsc.load_gather(adr, [rows, col])
                            al = asv + adv
                            al = jnp.where(al < 0, al * jnp.float32(0.2), al)
                            exv = jnp.exp(al - ch)
                            exb[h, pl.ds(q * 16, 16)] = exv
                            return 0

                        lax.fori_loop(0, _GB // 16, att_body, 0)

                    def msg_body(e, _):
                        lane = lax.iota(jnp.int32, 16)
                        esp = jnp.full((16,), e, jnp.int32)
                        ex0 = plsc.load_gather(exb, [jnp.zeros((16,), jnp.int32), esp])
                        ex1 = plsc.load_gather(
                            exb, [jnp.full((16,), 1, jnp.int32), esp])
                        ex2 = plsc.load_gather(
                            exb, [jnp.full((16,), 2, jnp.int32), esp])
                        exrow = (jnp.where(lane == 0, ex0, 0.0)
                                 + jnp.where(lane == 1, ex1, 0.0)
                                 + jnp.where(lane == 2, ex2, 0.0))
                        stg[e, pl.ds(HID, 16)] = exrow
                        for g in range(HID // 16):
                            exh = (ex0, ex1, ex2)[g // 4]
                            stg[e, pl.ds(g * 16, 16)] = (
                                hsr[e, pl.ds(g * 16, 16)] * exh)
                        return 0

                    lax.fori_loop(0, _GB, msg_body, 0)
                    pltpu.sync_copy(stg, acc.at[lidx], add=True)
                    return 0

                lax.fori_loop(0, nb, batch_body, 0)
                return 0

            lax.fori_loop(0, _TS // _SR, sub_body, 0)
            plsc.subcore_barrier()

            nwb = _R // 16

            def wr_body(kk, _):
                b = sid + kk * _NTILE

                @pl.when(b < nwb)
                def _():
                    pltpu.sync_copy(acc.at[pl.ds(b * 16, 16)],
                                    out_h.at[pl.ds(cbase + b * 16, 16)])
                return 0

            lax.fori_loop(0, (nwb + _NTILE - 1) // _NTILE, wr_body, 0)
            plsc.subcore_barrier()

    return k(srcs, dsts, hs, asrc, adst_pad, csplat)


def _att_fold(att):
    A = jnp.zeros((HID, 16), jnp.float32)
    for h in range(HEADS):
        A = A.at[h * NSTATE:(h + 1) * NSTATE, h].set(att[h])
    return A


def _csplat(asrc, adst):
    mx = jnp.max(asrc[:, :HEADS], axis=0) + jnp.max(adst[:, :HEADS], axis=0)
    C = jnp.where(mx < 0, mx * 0.2, mx)
    return jnp.repeat(C[:, None], 16, axis=1)


def kernel(x1, x2, edge_index, params):
    src = edge_index[0]
    dst = edge_index[1]
    types2d = x1[:, :1].astype(jnp.int32)

    g12 = params["gat12"]
    Asrc12 = _att_fold(g12["att_src"])
    Wdd12 = jnp.pad(g12["Wd"] @ _att_fold(g12["att_dst"])[:, :HEADS],
                    ((0, 0), (0, 13)))
    g21 = params["gat21"]
    Asrc21 = [_att_fold(p["att_src"]) for p in g21]
    Wdd21 = [jnp.pad(p["Wd"] @ _att_fold(p["att_dst"])[:, :HEADS],
                     ((0, 0), (0, 13))) for p in g21]
    tvs = [jnp.zeros((16,), jnp.float32).at[3].set(float(i))
           for i in range(NTYPES)]
    rnn2 = dict(params["rnn2"])
    rnn2["WihT"] = rnn2["Wih"].T
    rnn2["WhhT"] = rnn2["Whh"].T
    rnn1 = []
    for p in params["rnn1"]:
        q = dict(p)
        q["WihT"] = q["Wih"].T
        q["WhhT"] = q["Whh"].T
        rnn1.append(q)

    cols = jnp.arange(HID)
    Q = (jnp.arange(HEADS)[:, None] == (cols[None, :] // NSTATE)).astype(jnp.float32)
    P = ((cols[:, None] // HEADS) == jnp.arange(NSTATE)[None, :]).astype(jnp.float32)
    zpad = jnp.zeros((32, 16), jnp.float32)

    f1 = _enc1_pallas(params["enc1"], x1)
    h1 = f1
    h2 = _mlp_pallas(params["enc2"], x2, NSTATE)

    for _ in range(NSTEP):
        h1x = jnp.concatenate([h1, f1], axis=-1)
        hs12, as12 = _proj_pallas(h1x, g12["Ws"], Asrc12)
        ad12 = _adst_pallas(h2, Wdd12)
        acc2 = _edge_sc(src, dst, hs12, as12,
                        jnp.concatenate([ad12, zpad], axis=0),
                        _csplat(as12, ad12), 0)
        m2 = _pool_norm_pallas(acc2, g12["bias"], Q, P)

        hs21_l, as21_l = [], []
        for i in range(NTYPES):
            hsi, asi = _proj_pallas(h2, g21[i]["Ws"], Asrc21[i])
            hs21_l.append(hsi)
            as21_l.append(asi)
        hs21 = jnp.concatenate(hs21_l, axis=0)
        as21 = jnp.concatenate(as21_l, axis=0)
        ad21 = _adst_sel_pallas(h1, types2d, Wdd21, tvs)
        acc1 = _edge_sc(dst, src, hs21, as21,
                        jnp.concatenate([ad21, zpad], axis=0),
                        _csplat(as21, ad21), N2)
        m1 = _pool_norm_sel_pallas(acc1, types2d,
                                   [p["bias"] for p in g21], Q, P)

        h2 = _gru_pallas(rnn2, m2, h2)
        h1 = _gru1_pallas(rnn1, m1, h1, types2d)

    return _mlp_pallas(params["dec2"], h2, 8)


# speedup vs baseline: 3.6244x; 3.6244x over previous
import functools

import jax
import jax.numpy as jnp
from jax import lax
from jax.experimental import pallas as pl
from jax.experimental.pallas import tpu as pltpu
from jax.experimental.pallas import tpu_sc as plsc

N1 = 50000
N2 = 50000
E = 800000
NSTATE = 64
HEADS = 3
NTYPES = 3
NSTEP = 2
HID = HEADS * NSTATE

_BLK = 2000

_NSC = 2
_NTILE = 16
_R = 5408
_NCHUNK_PER_SC = 5
_OUT_N = _NSC * _NCHUNK_PER_SC * _R
_ROW = 208
_TS = E // _NTILE
_SR = 10000
_LD = 2000
_GB = 64
_CAP = 10240


def _mlp_block(x, W0, b0, W1, b1, W2, b2):
    h = jnp.maximum(x @ W0 + b0, 0.0)
    h = jnp.maximum(h @ W1 + b1, 0.0)
    return h @ W2 + b2


def _full(a):
    return pl.BlockSpec(a.shape, lambda i: (0,) * a.ndim)


def _mlp_kernel_body(x_ref, W0, b0, W1, b1, W2, b2, o_ref):
    o_ref[...] = _mlp_block(x_ref[...], W0[...], b0[...], W1[...], b1[...],
                            W2[...], b2[...])


def _mlp_pallas(p, x, nout):
    n = x.shape[0]
    args = (p["W0"], p["b0"], p["W1"], p["b1"], p["W2"], p["b2"])
    return pl.pallas_call(
        _mlp_kernel_body,
        grid=(n // _BLK,),
        in_specs=[pl.BlockSpec((_BLK, x.shape[1]), lambda i: (i, 0))]
        + [_full(a) for a in args],
        out_specs=pl.BlockSpec((_BLK, nout), lambda i: (i, 0)),
        out_shape=jax.ShapeDtypeStruct((n, nout), jnp.float32),
    )(x, *args)


def _enc1_body(x_ref, *rest):
    o_ref = rest[-1]
    x = x_ref[...]
    types = x[:, 0].astype(jnp.int32)
    out = jnp.zeros((x.shape[0], NSTATE), jnp.float32)
    for i in range(NTYPES):
        Wi = rest[i * 6:(i + 1) * 6]
        yi = _mlp_block(x, *[w[...] for w in Wi])
        out = jnp.where((types == i)[:, None], yi, out)
    o_ref[...] = out


def _enc1_pallas(ps, x):
    args = []
    for p in ps:
        args += [p["W0"], p["b0"], p["W1"], p["b1"], p["W2"], p["b2"]]
    return pl.pallas_call(
        _enc1_body,
        grid=(x.shape[0] // _BLK,),
        in_specs=[pl.BlockSpec((_BLK, x.shape[1]), lambda i: (i, 0))]
        + [_full(a) for a in args],
        out_specs=pl.BlockSpec((_BLK, NSTATE), lambda i: (i, 0)),
        out_shape=jax.ShapeDtypeStruct((x.shape[0], NSTATE), jnp.float32),
    )(x, *args)


def _proj_body(x_ref, Ws, Asrc, hs_ref, a_ref):
    hs = x_ref[...] @ Ws[...]
    hs_ref[...] = hs
    a_ref[...] = hs @ Asrc[...]


def _proj_pallas(x, Ws, Asrc):
    n = x.shape[0]
    return pl.pallas_call(
        _proj_body,
        grid=(n // _BLK,),
        in_specs=[pl.BlockSpec((_BLK, x.shape[1]), lambda i: (i, 0)),
                  _full(Ws), _full(Asrc)],
        out_specs=(pl.BlockSpec((_BLK, HID), lambda i: (i, 0)),
                   pl.BlockSpec((_BLK, 16), lambda i: (i, 0))),
        out_shape=(jax.ShapeDtypeStruct((n, HID), jnp.float32),
                   jax.ShapeDtypeStruct((n, 16), jnp.float32)),
    )(x, Ws, Asrc)


def _adst_body(x_ref, W, a_ref):
    a_ref[...] = x_ref[...] @ W[...]


def _adst_pallas(x, W):
    n = x.shape[0]
    return pl.pallas_call(
        _adst_body,
        grid=(n // _BLK,),
        in_specs=[pl.BlockSpec((_BLK, x.shape[1]), lambda i: (i, 0)), _full(W)],
        out_specs=pl.BlockSpec((_BLK, 16), lambda i: (i, 0)),
        out_shape=jax.ShapeDtypeStruct((n, 16), jnp.float32),
    )(x, W)


def _adst_sel_body(x_ref, t_ref, *rest):
    o_ref = rest[-1]
    x = x_ref[...]
    types = t_ref[...][:, 0]
    out = jnp.zeros((x.shape[0], 16), jnp.float32)
    for i in range(NTYPES):
        W, tv = rest[2 * i], rest[2 * i + 1]
        yi = x @ W[...] + tv[...]
        out = jnp.where((types == i)[:, None], yi, out)
    o_ref[...] = out


def _adst_sel_pallas(x, types2d, Ws, tvs):
    n = x.shape[0]
    args = []
    for W, tv in zip(Ws, tvs):
        args += [W, tv]
    return pl.pallas_call(
        _adst_sel_body,
        grid=(n // _BLK,),
        in_specs=[pl.BlockSpec((_BLK, x.shape[1]), lambda i: (i, 0)),
                  pl.BlockSpec((_BLK, 1), lambda i: (i, 0))]
        + [_full(a) for a in args],
        out_specs=pl.BlockSpec((_BLK, 16), lambda i: (i, 0)),
        out_shape=jax.ShapeDtypeStruct((n, 16), jnp.float32),
    )(x, types2d, *args)


def _gru_block(x, h, WihT, bih, WhhT, bhh):
    gi = x @ WihT + bih
    gh = h @ WhhT + bhh
    ir, iz, i_n = gi[:, :NSTATE], gi[:, NSTATE:2 * NSTATE], gi[:, 2 * NSTATE:]
    hr, hz, h_n = gh[:, :NSTATE], gh[:, NSTATE:2 * NSTATE], gh[:, 2 * NSTATE:]
    r = jax.nn.sigmoid(ir + hr)
    z = jax.nn.sigmoid(iz + hz)
    n = jnp.tanh(i_n + r * h_n)
    return (1.0 - z) * n + z * h


def _gru_body(x_ref, h_ref, WihT, bih, WhhT, bhh, o_ref):
    o_ref[...] = _gru_block(x_ref[...], h_ref[...], WihT[...], bih[...],
                            WhhT[...], bhh[...])


def _gru_pallas(p, x, h):
    n = x.shape[0]
    args = (p["WihT"], p["bih"], p["WhhT"], p["bhh"])
    return pl.pallas_call(
        _gru_body,
        grid=(n // _BLK,),
        in_specs=[pl.BlockSpec((_BLK, NSTATE), lambda i: (i, 0))] * 2
        + [_full(a) for a in args],
        out_specs=pl.BlockSpec((_BLK, NSTATE), lambda i: (i, 0)),
        out_shape=jax.ShapeDtypeStruct((n, NSTATE), jnp.float32),
    )(x, h, *args)


def _gru1_body(x_ref, h_ref, t_ref, *rest):
    o_ref = rest[-1]
    x, h = x_ref[...], h_ref[...]
    types = t_ref[...][:, 0]
    out = jnp.zeros_like(h)
    for i in range(NTYPES):
        Wi = [w[...] for w in rest[i * 4:(i + 1) * 4]]
        yi = _gru_block(x, h, *Wi)
        out = jnp.where((types == i)[:, None], yi, out)
    o_ref[...] = out


def _gru1_pallas(ps, x, h, types2d):
    n = x.shape[0]
    args = []
    for p in ps:
        args += [p["WihT"], p["bih"], p["WhhT"], p["bhh"]]
    return pl.pallas_call(
        _gru1_body,
        grid=(n // _BLK,),
        in_specs=[pl.BlockSpec((_BLK, NSTATE), lambda i: (i, 0))] * 2
        + [pl.BlockSpec((_BLK, 1), lambda i: (i, 0))]
        + [_full(a) for a in args],
        out_specs=pl.BlockSpec((_BLK, NSTATE), lambda i: (i, 0)),
        out_shape=jax.ShapeDtypeStruct((n, NSTATE), jnp.float32),
    )(x, h, types2d, *args)


def _pool_norm_body(agg_ref, den_ref, bias_ref, Q_ref, P_ref, o_ref):
    den192 = den_ref[...][:, :HEADS] @ Q_ref[...]
    m = agg_ref[...] / (den192 + 1e-30) + bias_ref[...]
    o_ref[...] = m @ P_ref[...]


def _pool_norm_pallas(agg, den, bias, Q, P):
    return pl.pallas_call(
        _pool_norm_body,
        grid=(N2 // _BLK,),
        in_specs=[pl.BlockSpec((_BLK, HID), lambda i: (i, 0)),
                  pl.BlockSpec((_BLK, 16), lambda i: (i, 0)),
                  _full(bias), _full(Q), _full(P)],
        out_specs=pl.BlockSpec((_BLK, NSTATE), lambda i: (i, 0)),
        out_shape=jax.ShapeDtypeStruct((N2, NSTATE), jnp.float32),
    )(agg, den, bias, Q, P)


def _pool_norm_sel_body(agg_ref, den_ref, t_ref, b0, b1, b2, Q_ref, P_ref,
                        o_ref):
    types = t_ref[...][:, 0]
    den192 = den_ref[...][:, :HEADS] @ Q_ref[...]
    bias = jnp.where((types == 0)[:, None], b0[...],
                     jnp.where((types == 1)[:, None], b1[...], b2[...]))
    m = agg_ref[...] / (den192 + 1e-30) + bias
    o_ref[...] = m @ P_ref[...]


def _pool_norm_sel_pallas(agg, den, types2d, biases, Q, P):
    return pl.pallas_call(
        _pool_norm_sel_body,
        grid=(N1 // _BLK,),
        in_specs=[pl.BlockSpec((_BLK, HID), lambda i: (i, 0)),
                  pl.BlockSpec((_BLK, 16), lambda i: (i, 0)),
                  pl.BlockSpec((_BLK, 1), lambda i: (i, 0))]
        + [_full(b) for b in biases] + [_full(Q), _full(P)],
        out_specs=pl.BlockSpec((_BLK, NSTATE), lambda i: (i, 0)),
        out_shape=jax.ShapeDtypeStruct((N1, NSTATE), jnp.float32),
    )(agg, den, types2d, *biases, Q, P)


def _edge_sc(srcs, dsts, hs, asrc, adst_pad, csplat, tstride):
    mesh = plsc.VectorSubcoreMesh(core_axis_name="c", subcore_axis_name="s",
                                  num_cores=_NSC, num_subcores=_NTILE)

    @functools.partial(
        pl.kernel,
        out_type=(jax.ShapeDtypeStruct((_OUT_N, HID), jnp.float32),
                  jax.ShapeDtypeStruct((_OUT_N, 16), jnp.float32)),
        mesh=mesh,
        compiler_params=pltpu.CompilerParams(
            needs_layout_passes=False, use_tc_tiling_on_sc=False),
        scratch_types=[
            pltpu.VMEM((_CAP,), jnp.int32),
            pltpu.VMEM((_CAP,), jnp.int32),
            pltpu.VMEM((_LD,), jnp.int32),
            pltpu.VMEM((_LD,), jnp.int32),
            pltpu.VMEM((_GB, HID), jnp.float32),
            pltpu.VMEM((_GB, HID), jnp.float32),
            pltpu.VMEM((_GB, 16), jnp.float32),
            pltpu.VMEM((_GB, 16), jnp.float32),
            pltpu.VMEM((_GB, 16), jnp.float32),
            pltpu.VMEM((_GB, 16), jnp.float32),
            pltpu.VMEM((_GB, 16), jnp.float32),
            pltpu.VMEM((HEADS, _GB), jnp.float32),
            pltpu.VMEM((_GB,), jnp.int32),
            pltpu.VMEM((_GB,), jnp.int32),
            pltpu.VMEM((_GB,), jnp.int32),
            pltpu.VMEM((_GB,), jnp.int32),
            pltpu.VMEM((_GB,), jnp.int32),
            pltpu.VMEM((8, HID), jnp.float32),
            pltpu.VMEM((8, 16), jnp.float32),
            pltpu.VMEM((HEADS, 16), jnp.float32),
            pltpu.VMEM_SHARED((_R + 16, HID), jnp.float32),
            pltpu.VMEM_SHARED((_R + 16, 16), jnp.float32),
            pltpu.SemaphoreType.DMA,
            pltpu.SemaphoreType.DMA,
            pltpu.SemaphoreType.DMA,
            pltpu.SemaphoreType.DMA,
            pltpu.SemaphoreType.DMA,
            pltpu.SemaphoreType.DMA,
            pltpu.SemaphoreType.DMA,
            pltpu.SemaphoreType.DMA,
        ],
    )
    def k(srcs_h, dsts_h, hs_h, asrc_h, adst_h, c_h, outM_h, outD_h,
          sC, dC, sL, dL, hsrA, hsrB, asr, adrA, adrB, exdA, exdB, exb,
          gidx, fidxA, fidxB, lidxA, lidxB, zM, zD, cbuf, accM, accD,
          semAA, semAB, semG, semH, semSA, semSB, semDA, semDB):
        cid = lax.axis_index("c")
        sid = lax.axis_index("s")
        ebase = sid * _TS

        setA = (fidxA, adrA, lidxA, hsrA, exdA, semAA, semSA, semDA)
        setB = (fidxB, adrB, lidxB, hsrB, exdB, semAB, semSB, semDB)

        pltpu.sync_copy(c_h, cbuf)
        for r in range(8):
            for g in range(HID // 16):
                zM[r, pl.ds(g * 16, 16)] = jnp.zeros((16,), jnp.float32)
            zD[r, pl.ds(0, 16)] = jnp.zeros((16,), jnp.float32)

        def issue_adst(off, st, cbase):
            fx, adx, _, _, _, semA, _, _ = st

            def ib(q, _):
                lv = dC[pl.ds(off + q * 16, 16)]
                fx[pl.ds(q * 16, 16)] = jnp.minimum(
                    lv + jnp.full((16,), cbase, jnp.int32),
                    jnp.full((16,), N1 - 1, jnp.int32))
                return 0

            lax.fori_loop(0, _GB // 16, ib, 0)
            pltpu.async_copy(adst_h.at[fx], adx, semA)

        def do_batch(off, st, pend):
            fx, adx, lx, hx, ex_d, semA, semS, semD = st

            @pl.when(pend == 1)
            def _():
                pltpu.make_async_copy(hx, accM.at[lx], semS).wait()
                pltpu.make_async_copy(ex_d, accD.at[lx], semD).wait()

            def ib(q, _):
                gidx[pl.ds(q * 16, 16)] = sC[pl.ds(off + q * 16, 16)]
                lx[pl.ds(q * 16, 16)] = dC[pl.ds(off + q * 16, 16)]
                return 0

            lax.fori_loop(0, _GB // 16, ib, 0)
            pltpu.make_async_copy(adst_h.at[fx], adx, semA).wait()

            if tstride:
                def tmix_body(q, _):
                    t = plsc.load_gather(
                        adx,
                        [q * 16 + lax.iota(jnp.int32, 16),
                         jnp.full((16,), 3, jnp.int32)])
                    ti = t.astype(jnp.int32)
                    sv = gidx[pl.ds(q * 16, 16)]
                    gidx[pl.ds(q * 16, 16)] = ti * tstride + sv
                    return 0

                lax.fori_loop(0, _GB // 16, tmix_body, 0)

            g1 = pltpu.async_copy(asrc_h.at[gidx], asr, semG)
            g2 = pltpu.async_copy(hs_h.at[gidx], hx, semH)
            g1.wait()
            g2.wait()

            for h in range(HEADS):
                def att_body(q, _):
                    ch = cbuf[h, :]
                    rows = q * 16 + lax.iota(jnp.int32, 16)
                    col = jnp.full((16,), h, jnp.int32)
                    asv = plsc.load_gather(asr, [rows, col])
                    adv = plsc.load_gather(adx, [rows, col])
                    al = asv + adv
                    al = jnp.where(al < 0, al * jnp.float32(0.2), al)
                    exv = jnp.exp(al - ch)
                    exb[h, pl.ds(q * 16, 16)] = exv
                    return 0

                lax.fori_loop(0, _GB // 16, att_body, 0)

            def msg_body(e, _):
                lane = lax.iota(jnp.int32, 16)
                esp = jnp.full((16,), e, jnp.int32)
                ex0 = plsc.load_gather(exb, [jnp.zeros((16,), jnp.int32), esp])
                ex1 = plsc.load_gather(exb, [jnp.full((16,), 1, jnp.int32), esp])
                ex2 = plsc.load_gather(exb, [jnp.full((16,), 2, jnp.int32), esp])
                exrow = (jnp.where(lane == 0, ex0, 0.0)
                         + jnp.where(lane == 1, ex1, 0.0)
                         + jnp.where(lane == 2, ex2, 0.0))
                ex_d[e, pl.ds(0, 16)] = exrow
                for g in range(HID // 16):
                    exh = (ex0, ex1, ex2)[g // 4]
                    hx[e, pl.ds(g * 16, 16)] = hx[e, pl.ds(g * 16, 16)] * exh
                return 0

            lax.fori_loop(0, _GB, msg_body, 0)
            pltpu.async_copy(hx, accM.at[lx], semS, add=True)
            pltpu.async_copy(ex_d, accD.at[lx], semD, add=True)
            return jnp.int32(1)

        for j in range(_NCHUNK_PER_SC):
            cbase = (cid * _NCHUNK_PER_SC + j) * _R

            nzb = (_R + 16) // 8

            def zero_body(kk, _):
                b = sid + kk * _NTILE

                @pl.when(b < nzb)
                def _():
                    pltpu.sync_copy(zM, accM.at[pl.ds(b * 8, 8)])
                    pltpu.sync_copy(zD, accD.at[pl.ds(b * 8, 8)])
                return 0

            lax.fori_loop(0, (nzb + _NTILE - 1) // _NTILE, zero_body, 0)
            plsc.subcore_barrier()

            def sub_body(sub, pends):
                pendA, pendB = pends
                sbase = ebase + sub * _SR

                def load_body(b, cur):
                    pltpu.sync_copy(srcs_h.at[pl.ds(sbase + b * _LD, _LD)], sL)
                    pltpu.sync_copy(dsts_h.at[pl.ds(sbase + b * _LD, _LD)], dL)

                    def scan_body(g, cur):
                        s16 = sL[pl.ds(g * 16, 16)]
                        d16 = dL[pl.ds(g * 16, 16)]
                        loc = d16 - jnp.full((16,), cbase, jnp.int32)
                        own = (loc >= 0) & (loc < _R)
                        inc = plsc.cumsum(own.astype(jnp.int32))
                        pos = jnp.maximum(cur + inc - 1, 0)
                        plsc.store_scatter(sC, [pos], s16, mask=own)
                        plsc.store_scatter(dC, [pos], loc, mask=own)
                        return cur + jnp.sum(own.astype(jnp.int32))

                    return lax.fori_loop(0, _LD // 16, scan_body, cur)

                cur = lax.fori_loop(0, _SR // _LD, load_body, jnp.int32(0))

                npair = (cur + 2 * _GB - 1) // (2 * _GB)
                for kk in range(2 * _GB // 16):
                    idx = cur + kk * 16 + lax.iota(jnp.int32, 16)
                    pm = idx < npair * 2 * _GB
                    plsc.store_scatter(sC, [idx], jnp.zeros((16,), jnp.int32),
                                       mask=pm)
                    plsc.store_scatter(dC, [idx],
                                       jnp.full((16,), _R, jnp.int32), mask=pm)

                @pl.when(npair > 0)
                def _():
                    issue_adst(0, setA, cbase)

                def pair_body(k2, pends):
                    pendA, pendB = pends
                    off0 = 2 * k2 * _GB
                    issue_adst(off0 + _GB, setB, cbase)
                    pendA = do_batch(off0, setA, pendA)

                    @pl.when(k2 + 1 < npair)
                    def _():
                        issue_adst(off0 + 2 * _GB, setA, cbase)

                    pendB = do_batch(off0 + _GB, setB, pendB)
                    return (pendA, pendB)

                return lax.fori_loop(0, npair, pair_body, (pendA, pendB))

            pendA, pendB = lax.fori_loop(
                0, _TS // _SR, sub_body, (jnp.int32(0), jnp.int32(0)))

            @pl.when(pendA == 1)
            def _():
                pltpu.make_async_copy(hsrA, accM.at[lidxA], semSA).wait()
                pltpu.make_async_copy(exdA, accD.at[lidxA], semDA).wait()

            @pl.when(pendB == 1)
            def _():
                pltpu.make_async_copy(hsrB, accM.at[lidxB], semSB).wait()
                pltpu.make_async_copy(exdB, accD.at[lidxB], semDB).wait()

            plsc.subcore_barrier()

            nwb = _R // 16

            def wr_body(kk, _):
                b = sid + kk * _NTILE

                @pl.when(b < nwb)
                def _():
                    pltpu.sync_copy(accM.at[pl.ds(b * 16, 16)],
                                    outM_h.at[pl.ds(cbase + b * 16, 16)])
                    pltpu.sync_copy(accD.at[pl.ds(b * 16, 16)],
                                    outD_h.at[pl.ds(cbase + b * 16, 16)])
                return 0

            lax.fori_loop(0, (nwb + _NTILE - 1) // _NTILE, wr_body, 0)
            plsc.subcore_barrier()

    return k(srcs, dsts, hs, asrc, adst_pad, csplat)


def _att_fold(att):
    A = jnp.zeros((HID, 16), jnp.float32)
    for h in range(HEADS):
        A = A.at[h * NSTATE:(h + 1) * NSTATE, h].set(att[h])
    return A


def _csplat(asrc, adst):
    mx = jnp.max(asrc[:, :HEADS], axis=0) + jnp.max(adst[:, :HEADS], axis=0)
    C = jnp.where(mx < 0, mx * 0.2, mx)
    return jnp.repeat(C[:, None], 16, axis=1)


def kernel(x1, x2, edge_index, params):
    src = edge_index[0]
    dst = edge_index[1]
    types2d = x1[:, :1].astype(jnp.int32)

    g12 = params["gat12"]
    Asrc12 = _att_fold(g12["att_src"])
    Wdd12 = jnp.pad(g12["Wd"] @ _att_fold(g12["att_dst"])[:, :HEADS],
                    ((0, 0), (0, 13)))
    g21 = params["gat21"]
    Asrc21 = [_att_fold(p["att_src"]) for p in g21]
    Wdd21 = [jnp.pad(p["Wd"] @ _att_fold(p["att_dst"])[:, :HEADS],
                     ((0, 0), (0, 13))) for p in g21]
    tvs = [jnp.zeros((16,), jnp.float32).at[3].set(float(i))
           for i in range(NTYPES)]
    rnn2 = dict(params["rnn2"])
    rnn2["WihT"] = rnn2["Wih"].T
    rnn2["WhhT"] = rnn2["Whh"].T
    rnn1 = []
    for p in params["rnn1"]:
        q = dict(p)
        q["WihT"] = q["Wih"].T
        q["WhhT"] = q["Whh"].T
        rnn1.append(q)

    cols = jnp.arange(HID)
    Q = (jnp.arange(HEADS)[:, None] == (cols[None, :] // NSTATE)).astype(jnp.float32)
    P = ((cols[:, None] // HEADS) == jnp.arange(NSTATE)[None, :]).astype(jnp.float32)
    zpad = jnp.zeros((32, 16), jnp.float32)

    f1 = _enc1_pallas(params["enc1"], x1)
    h1 = f1
    h2 = _mlp_pallas(params["enc2"], x2, NSTATE)

    for _ in range(NSTEP):
        h1x = jnp.concatenate([h1, f1], axis=-1)
        hs12, as12 = _proj_pallas(h1x, g12["Ws"], Asrc12)
        ad12 = _adst_pallas(h2, Wdd12)
        agg2, den2 = _edge_sc(src, dst, hs12, as12,
                              jnp.concatenate([ad12, zpad], axis=0),
                              _csplat(as12, ad12), 0)
        m2 = _pool_norm_pallas(agg2, den2, g12["bias"], Q, P)

        hs21_l, as21_l = [], []
        for i in range(NTYPES):
            hsi, asi = _proj_pallas(h2, g21[i]["Ws"], Asrc21[i])
            hs21_l.append(hsi)
            as21_l.append(asi)
        hs21 = jnp.concatenate(hs21_l, axis=0)
        as21 = jnp.concatenate(as21_l, axis=0)
        ad21 = _adst_sel_pallas(h1, types2d, Wdd21, tvs)
        agg1, den1 = _edge_sc(dst, src, hs21, as21,
                              jnp.concatenate([ad21, zpad], axis=0),
                              _csplat(as21, ad21), N2)
        m1 = _pool_norm_sel_pallas(agg1, den1, types2d,
                                   [p["bias"] for p in g21], Q, P)

        h2 = _gru_pallas(rnn2, m2, h2)
        h1 = _gru1_pallas(rnn1, m1, h1, types2d)

    return _mlp_pallas(params["dec2"], h2, 8)
